# baseline (speedup 1.0000x reference)
import functools

import jax
import jax.numpy as jnp
from jax import lax
from jax.experimental import pallas as pl
from jax.experimental.pallas import tpu as pltpu
from jax.experimental.pallas import tpu_sc as plsc

_D_MODEL = 2048
_N_EXP = 64
_TOP_K = 2
_BLK = 2048

_NC = 2
_NS = 16
_NW = _NC * _NS


def _probs_body(x_ref, w_ref, probs_ref):
    x = x_ref[...]
    w = w_ref[...]
    logits = jax.lax.dot_general(
        x, w, (((1,), (1,)), ((), ())), preferred_element_type=jnp.float32
    )
    m1 = jnp.max(logits, axis=1, keepdims=True)
    e = jnp.exp(logits - m1)
    s = jnp.sum(e, axis=1, keepdims=True)
    probs_ref[...] = e / s


def _probs_call(xf, W):
    T, D = xf.shape
    return pl.pallas_call(
        _probs_body,
        grid=(T // _BLK,),
        in_specs=[
            pl.BlockSpec((_BLK, D), lambda i: (i, 0)),
            pl.BlockSpec((_N_EXP, D), lambda i: (0, 0)),
        ],
        out_specs=pl.BlockSpec((_BLK, _N_EXP), lambda i: (i, 0)),
        out_shape=jax.ShapeDtypeStruct((T, _N_EXP), jnp.float32),
        compiler_params=pltpu.CompilerParams(
            dimension_semantics=("parallel",),
        ),
    )(xf, W)


def _sc_topk_call(probs, T):
    tpw = T // _NW

    @functools.partial(
        pl.kernel,
        out_type=[
            jax.ShapeDtypeStruct((T * _TOP_K,), jnp.int32),
            jax.ShapeDtypeStruct((T * _TOP_K,), jnp.float32),
        ],
        mesh=plsc.VectorSubcoreMesh(core_axis_name="c", subcore_axis_name="s"),
        compiler_params=pltpu.CompilerParams(needs_layout_passes=False),
        scratch_types=[
            pltpu.VMEM((tpw, _N_EXP), jnp.float32),
            pltpu.VMEM((tpw * _TOP_K,), jnp.int32),
            pltpu.VMEM((tpw * _TOP_K,), jnp.float32),
        ],
    )
    def sc_topk(probs_hbm, idx_hbm, wts_hbm, p_v, idx_v, wts_v):
        wid = lax.axis_index("s") * _NC + lax.axis_index("c")
        base = wid * tpw
        pltpu.sync_copy(probs_hbm.at[pl.ds(base, tpw)], p_v)

        iota = lax.iota(jnp.int32, 16)
        tags = [63 - (iota + 16 * j) for j in range(4)]
        mask_hi = jnp.full((16,), ~jnp.int32(63))
        int_min = jnp.full((16,), jnp.int32(-2147483648))
        zero_i = jnp.zeros((16,), jnp.int32)
        zero_f = jnp.zeros((16,), jnp.float32)

        def bcast_max(v):
            return plsc.cummax(jnp.flip(plsc.cummax(v)))

        def top2_token(t):
            r = [p_v[t, pl.ds(16 * j, 16)] for j in range(4)]
            k = [
                (plsc.bitcast(r[j], jnp.int32) & mask_hi) | tags[j]
                for j in range(4)
            ]
            kt = jnp.maximum(jnp.maximum(k[0], k[1]), jnp.maximum(k[2], k[3]))
            k1 = bcast_max(kt)
            km = [jnp.where(k[j] == k1, int_min, k[j]) for j in range(4)]
            kt2 = jnp.maximum(
                jnp.maximum(km[0], km[1]), jnp.maximum(km[2], km[3])
            )
            k2 = bcast_max(kt2)
            i1 = 63 - (k1 & 63)
            i2 = 63 - (k2 & 63)
            p1 = plsc.bitcast(k1 & mask_hi, jnp.float32)
            p2 = plsc.bitcast(k2 & mask_hi, jnp.float32)
            s = p1 + p2
            return i1, i2, p1 / s, p2 / s

        def body(g, _):
            acc_i = zero_i
            acc_w = zero_f
            for j in range(8):
                i1, i2, w1, w2 = top2_token(g * 8 + j)
                acc_i = jnp.where(iota == 2 * j, i1, acc_i)
                acc_i = jnp.where(iota == 2 * j + 1, i2, acc_i)
                acc_w = jnp.where(iota == 2 * j, w1, acc_w)
                acc_w = jnp.where(iota == 2 * j + 1, w2, acc_w)
            idx_v[pl.ds(g * 16, 16)] = acc_i
            wts_v[pl.ds(g * 16, 16)] = acc_w
            return 0

        lax.fori_loop(0, tpw // 8, body, 0)
        pltpu.sync_copy(idx_v, idx_hbm.at[pl.ds(base * _TOP_K, tpw * _TOP_K)])
        pltpu.sync_copy(wts_v, wts_hbm.at[pl.ds(base * _TOP_K, tpw * _TOP_K)])

    return sc_topk(probs)


def kernel(x, W):
    B, S, D = x.shape
    T = B * S
    xf = x.reshape(T, D)
    n_chunks = 4
    tc = T // n_chunks
    probs_l, idx_l, wts_l = [], [], []
    for ci in range(n_chunks):
        probs_c = _probs_call(xf[ci * tc:(ci + 1) * tc], W)
        idx_c, wts_c = _sc_topk_call(probs_c, tc)
        probs_l.append(probs_c)
        idx_l.append(idx_c)
        wts_l.append(wts_c)
    probs = jnp.concatenate(probs_l, axis=0)
    idx_flat = jnp.concatenate(idx_l, axis=0)
    wts_flat = jnp.concatenate(wts_l, axis=0)
    return (
        probs.reshape(B, S, _N_EXP),
        idx_flat.reshape(B, S, _TOP_K),
        wts_flat.reshape(B, S, _TOP_K),
    )


# speedup vs baseline: 2.8514x; 2.8514x over previous
import jax
import jax.numpy as jnp
from jax.experimental import pallas as pl
from jax.experimental.pallas import tpu as pltpu

_D_MODEL = 2048
_N_EXP = 64
_TOP_K = 2
_BLK = 2048


def _router_body(x_ref, w_ref, probs_ref, idx_ref, wts_ref):
    x = x_ref[...]
    w = w_ref[...]
    logits = jax.lax.dot_general(
        x, w, (((1,), (1,)), ((), ())), preferred_element_type=jnp.float32
    )
    eidx = jax.lax.broadcasted_iota(jnp.int32, logits.shape, 1)
    bits = jax.lax.bitcast_convert_type(logits, jnp.int32)
    keys = jnp.where(bits < 0, bits ^ jnp.int32(0x7FFFFFFF), bits)
    keys = (keys & jnp.int32(~63)) | (63 - eidx)
    k1 = jnp.max(keys, axis=1, keepdims=True)
    k2 = jnp.max(jnp.where(keys == k1, jnp.int32(-2147483648), keys), axis=1,
                 keepdims=True)
    i1 = 63 - (k1 & 63)
    i2 = 63 - (k2 & 63)
    v1 = k1 & jnp.int32(~63)
    v2 = k2 & jnp.int32(~63)
    m1 = jax.lax.bitcast_convert_type(
        jnp.where(v1 < 0, v1 ^ jnp.int32(0x7FFFFFFF), v1), jnp.float32)
    m2 = jax.lax.bitcast_convert_type(
        jnp.where(v2 < 0, v2 ^ jnp.int32(0x7FFFFFFF), v2), jnp.float32)

    e = jnp.exp(logits - m1)
    s = jnp.sum(e, axis=1, keepdims=True)
    probs_ref[...] = e / s

    t = jnp.exp(m2 - m1)
    w1 = 1.0 / (1.0 + t)
    w2 = t / (1.0 + t)
    idx_ref[...] = jnp.concatenate([i1, i2], axis=1)
    wts_ref[...] = jnp.concatenate([w1, w2], axis=1)


def _router(xf, W, interpret=False):
    T, D = xf.shape
    return pl.pallas_call(
        _router_body,
        grid=(T // _BLK,),
        in_specs=[
            pl.BlockSpec((_BLK, D), lambda i: (i, 0)),
            pl.BlockSpec((_N_EXP, D), lambda i: (0, 0)),
        ],
        out_specs=[
            pl.BlockSpec((_BLK, _N_EXP), lambda i: (i, 0)),
            pl.BlockSpec((_BLK, _TOP_K), lambda i: (i, 0)),
            pl.BlockSpec((_BLK, _TOP_K), lambda i: (i, 0)),
        ],
        out_shape=[
            jax.ShapeDtypeStruct((T, _N_EXP), jnp.float32),
            jax.ShapeDtypeStruct((T, _TOP_K), jnp.int32),
            jax.ShapeDtypeStruct((T, _TOP_K), jnp.float32),
        ],
        compiler_params=pltpu.CompilerParams(
            dimension_semantics=("parallel",),
        ),
        interpret=interpret,
    )(xf, W)


def kernel(x, W):
    B, S, D = x.shape
    T = B * S
    probs, idx, wts = _router(x.reshape(T, D), W)
    return (
        probs.reshape(B, S, _N_EXP),
        idx.reshape(B, S, _TOP_K),
        wts.reshape(B, S, _TOP_K),
    )


# speedup vs baseline: 2.8740x; 1.0079x over previous
import jax
import jax.numpy as jnp
from jax.experimental import pallas as pl
from jax.experimental.pallas import tpu as pltpu

_D_MODEL = 2048
_N_EXP = 64
_TOP_K = 2
_BLK = 2048


def _router_body(x_ref, w_ref, probs_ref, idx_ref, wts_ref):
    x = x_ref[...]
    w = w_ref[...]
    logits = jax.lax.dot_general(
        x, w, (((1,), (1,)), ((), ())), preferred_element_type=jnp.float32
    )
    m1 = jnp.max(logits, axis=1, keepdims=True)
    e = jnp.exp(logits - m1)
    s = jnp.sum(e, axis=1, keepdims=True)
    probs_ref[...] = e * (1.0 / s)

    eidx = jax.lax.broadcasted_iota(jnp.int32, logits.shape, 1)
    i1 = jnp.min(jnp.where(logits == m1, eidx, _N_EXP), axis=1, keepdims=True)
    masked = jnp.where(eidx == i1, -jnp.inf, logits)
    m2 = jnp.max(masked, axis=1, keepdims=True)
    i2 = jnp.min(jnp.where(masked == m2, eidx, _N_EXP), axis=1, keepdims=True)

    t = jnp.exp(m2 - m1)
    w1 = 1.0 / (1.0 + t)
    w2 = t / (1.0 + t)
    idx_ref[...] = jnp.concatenate([i1, i2], axis=1)
    wts_ref[...] = jnp.concatenate([w1, w2], axis=1)


def _router(xf, W, interpret=False):
    T, D = xf.shape
    return pl.pallas_call(
        _router_body,
        grid=(T // _BLK,),
        in_specs=[
            pl.BlockSpec((_BLK, D), lambda i: (i, 0)),
            pl.BlockSpec((_N_EXP, D), lambda i: (0, 0)),
        ],
        out_specs=[
            pl.BlockSpec((_BLK, _N_EXP), lambda i: (i, 0)),
            pl.BlockSpec((_BLK, _TOP_K), lambda i: (i, 0)),
            pl.BlockSpec((_BLK, _TOP_K), lambda i: (i, 0)),
        ],
        out_shape=[
            jax.ShapeDtypeStruct((T, _N_EXP), jnp.float32),
            jax.ShapeDtypeStruct((T, _TOP_K), jnp.int32),
            jax.ShapeDtypeStruct((T, _TOP_K), jnp.float32),
        ],
        compiler_params=pltpu.CompilerParams(
            dimension_semantics=("parallel",),
        ),
        interpret=interpret,
    )(xf, W)


def kernel(x, W):
    B, S, D = x.shape
    T = B * S
    probs, idx, wts = _router(x.reshape(T, D), W)
    return (
        probs.reshape(B, S, _N_EXP),
        idx.reshape(B, S, _TOP_K),
        wts.reshape(B, S, _TOP_K),
    )
